# baseline (speedup 1.0000x reference)
import functools

import jax
import jax.numpy as jnp
from jax import lax
from jax.experimental import pallas as pl
from jax.experimental.pallas import tpu as pltpu
from jax.experimental.pallas import tpu_sc as plsc

VOCAB = 1000000
BATCH = 4096
SEQ_LEN = 200
TIME_DIM = 32

NUM_CORES = 2
NUM_SUBCORES = 16
NW = NUM_CORES * NUM_SUBCORES
BT = BATCH // NW
NBUF = 2
SP = 1
NCH = SEQ_LEN // SP


def _relayout_tc(table_t):
    CH = 2048

    def body(in_ref, out_ref):
        out_ref[:, :TIME_DIM] = in_ref[...].T

    return pl.pallas_call(
        body,
        grid=((VOCAB + CH - 1) // CH,),
        in_specs=[pl.BlockSpec((TIME_DIM, CH), lambda i: (0, i))],
        out_specs=pl.BlockSpec((CH, 128), lambda i: (i, 0)),
        out_shape=jax.ShapeDtypeStruct((VOCAB, 128), jnp.float32),
    )(table_t)


def _gather_sc(table, idx):
    mesh = plsc.VectorSubcoreMesh(core_axis_name="c", subcore_axis_name="s")

    @functools.partial(
        pl.kernel,
        mesh=mesh,
        compiler_params=pltpu.CompilerParams(
            use_tc_tiling_on_sc=False, needs_layout_passes=False),
        out_type=jax.ShapeDtypeStruct((SEQ_LEN, 4, NW, 8, 128), jnp.float32),
        scratch_types=[
            pltpu.VMEM((BT, SEQ_LEN), jnp.int32),
            pltpu.VMEM((SEQ_LEN * BT,), jnp.int32),
            pltpu.VMEM((NBUF, SP * BT, 128), jnp.float32),
            pltpu.VMEM((NBUF, SP, 4, 8, 128), jnp.float32),
            pltpu.SemaphoreType.DMA((NBUF,)),
            pltpu.SemaphoreType.DMA((NBUF,)),
        ],
    )
    def k(table_hbm, idx_hbm, out_hbm, idx2, srow, rows, obuf, gsem, osem):
        wid = lax.axis_index("s") * NUM_CORES + lax.axis_index("c")
        base = wid * BT
        pltpu.sync_copy(idx_hbm.at[pl.ds(base, BT)], idx2)

        lane = lax.iota(jnp.int32, 16)
        cvecs = [lane + 16 * cb for cb in range(8)]

        def tbody(s, carry):
            svec = jnp.full((16,), 0, jnp.int32) + s
            vals = [plsc.load_gather(idx2, [cvecs[cb], svec]) for cb in range(8)]
            for cb in range(8):
                srow.at[pl.ds(s * BT + 16 * cb, 16)][...] = vals[cb]
            return carry

        lax.fori_loop(0, SEQ_LEN, tbody, 0)

        def g_copy(p, b):
            return pltpu.make_async_copy(
                table_hbm.at[srow.at[pl.ds(p * SP * BT, SP * BT)]],
                rows.at[b], gsem.at[b])

        def o_copy(p, b):
            return pltpu.make_async_copy(
                obuf.at[b], out_hbm.at[pl.ds(p * SP, SP), :, wid], osem.at[b])

        svecs = [[lane + 16 * cb + BT * sh for cb in range(8)] for sh in range(SP)]

        def extract(p, b):
            rbuf = rows.at[b]
            for sh in range(SP):
                for cb in range(8):
                    for dh in range(2):
                        vals = [
                            plsc.load_gather(
                                rbuf,
                                [svecs[sh][cb],
                                 jnp.full((16,), 16 * dh + i, jnp.int32)])
                            for i in range(16)
                        ]
                        for i in range(16):
                            d = 16 * dh + i
                            obuf.at[b, sh, d // 8, d % 8,
                                    pl.ds(16 * cb, 16)][...] = vals[i]

        g_copy(0, 0).start()

        def body(jj, carry):
            for b in range(NBUF):
                p = jj * NBUF + b
                nb = (b + 1) % NBUF

                @pl.when(p + 1 < NCH)
                def _():
                    @pl.when(p >= 1)
                    def _():
                        o_copy(p - 1, nb).wait()

                    g_copy(p + 1, nb).start()

                g_copy(p, b).wait()
                extract(p, b)
                o_copy(p, b).start()
            return carry

        lax.fori_loop(0, NCH // NBUF, body, 0)
        o_copy(NCH - 2, 0).wait()
        o_copy(NCH - 1, 1).wait()

    return k(table, idx)


def kernel(token_ids, time_embeddings):
    table_p = _relayout_tc(time_embeddings.T)
    out5 = _gather_sc(table_p, token_ids)
    return out5.transpose(2, 4, 0, 1, 3).reshape(BATCH, SEQ_LEN, TIME_DIM)


# speedup vs baseline: 1.2761x; 1.2761x over previous
import functools

import jax
import jax.numpy as jnp
from jax import lax
from jax.experimental import pallas as pl
from jax.experimental.pallas import tpu as pltpu
from jax.experimental.pallas import tpu_sc as plsc

VOCAB = 1000000
BATCH = 4096
SEQ_LEN = 200
TIME_DIM = 32

NUM_CORES = 2
NUM_SUBCORES = 16
NW = NUM_CORES * NUM_SUBCORES
BT = BATCH // NW
NBUF = 2
SP = 1
NCH = SEQ_LEN // SP


def _relayout_tc(table_t):
    CH = 8192

    def body(in_ref, out_ref):
        eye = jnp.eye(TIME_DIM, dtype=jnp.float32)
        out_ref[:, :TIME_DIM] = jax.lax.dot_general(
            in_ref[...], eye, (((0,), (0,)), ((), ())),
            preferred_element_type=jnp.float32)

    return pl.pallas_call(
        body,
        grid=((VOCAB + CH - 1) // CH,),
        in_specs=[pl.BlockSpec((TIME_DIM, CH), lambda i: (0, i))],
        out_specs=pl.BlockSpec((CH, 128), lambda i: (i, 0)),
        out_shape=jax.ShapeDtypeStruct((VOCAB, 128), jnp.float32),
    )(table_t)


def _gather_sc(table, idx):
    mesh = plsc.VectorSubcoreMesh(core_axis_name="c", subcore_axis_name="s")

    @functools.partial(
        pl.kernel,
        mesh=mesh,
        compiler_params=pltpu.CompilerParams(
            use_tc_tiling_on_sc=False, needs_layout_passes=False),
        out_type=jax.ShapeDtypeStruct((SEQ_LEN, 4, NW, 8, 128), jnp.float32),
        scratch_types=[
            pltpu.VMEM((BT, SEQ_LEN), jnp.int32),
            pltpu.VMEM((SEQ_LEN * BT,), jnp.int32),
            pltpu.VMEM((NBUF, SP * BT, 128), jnp.float32),
            pltpu.VMEM((NBUF, SP, 4, 8, 128), jnp.float32),
            pltpu.SemaphoreType.DMA((NBUF,)),
            pltpu.SemaphoreType.DMA((NBUF,)),
        ],
    )
    def k(table_hbm, idx_hbm, out_hbm, idx2, srow, rows, obuf, gsem, osem):
        wid = lax.axis_index("s") * NUM_CORES + lax.axis_index("c")
        base = wid * BT
        pltpu.sync_copy(idx_hbm.at[pl.ds(base, BT)], idx2)

        lane = lax.iota(jnp.int32, 16)
        cvecs = [lane + 16 * cb for cb in range(8)]

        def tbody(s, carry):
            svec = jnp.full((16,), 0, jnp.int32) + s
            vals = [plsc.load_gather(idx2, [cvecs[cb], svec]) for cb in range(8)]
            for cb in range(8):
                srow.at[pl.ds(s * BT + 16 * cb, 16)][...] = vals[cb]
            return carry

        lax.fori_loop(0, SEQ_LEN, tbody, 0)

        def g_copy(p, b):
            return pltpu.make_async_copy(
                table_hbm.at[srow.at[pl.ds(p * SP * BT, SP * BT)]],
                rows.at[b], gsem.at[b])

        def o_copy(p, b):
            return pltpu.make_async_copy(
                obuf.at[b], out_hbm.at[pl.ds(p * SP, SP), :, wid], osem.at[b])

        svecs = [[lane + 16 * cb + BT * sh for cb in range(8)] for sh in range(SP)]

        def extract(p, b):
            rbuf = rows.at[b]
            for sh in range(SP):
                for cb in range(8):
                    for dh in range(2):
                        vals = [
                            plsc.load_gather(
                                rbuf,
                                [svecs[sh][cb],
                                 jnp.full((16,), 16 * dh + i, jnp.int32)])
                            for i in range(16)
                        ]
                        for i in range(16):
                            d = 16 * dh + i
                            obuf.at[b, sh, d // 8, d % 8,
                                    pl.ds(16 * cb, 16)][...] = vals[i]

        g_copy(0, 0).start()

        def body(jj, carry):
            for b in range(NBUF):
                p = jj * NBUF + b
                nb = (b + 1) % NBUF

                @pl.when(p + 1 < NCH)
                def _():
                    @pl.when(p >= 1)
                    def _():
                        o_copy(p - 1, nb).wait()

                    g_copy(p + 1, nb).start()

                g_copy(p, b).wait()
                extract(p, b)
                o_copy(p, b).start()
            return carry

        lax.fori_loop(0, NCH // NBUF, body, 0)
        o_copy(NCH - 2, 0).wait()
        o_copy(NCH - 1, 1).wait()

    return k(table, idx)


def kernel(token_ids, time_embeddings):
    table_p = _relayout_tc(time_embeddings.T)
    out5 = _gather_sc(table_p, token_ids)
    return out5.transpose(2, 4, 0, 1, 3).reshape(BATCH, SEQ_LEN, TIME_DIM)
